# Initial kernel scaffold; baseline (speedup 1.0000x reference)
import jax
import jax.numpy as jnp
from jax.experimental import pallas as pl


def kernel(x, router_W, gate_W, up_W, down_W):
    raise NotImplementedError("write your pallas kernel here")


# speedup vs baseline: 1.1313x; 1.1313x over previous
import functools

import jax
import jax.numpy as jnp
from jax.experimental import pallas as pl
from jax.experimental.pallas import tpu as pltpu

E = 8
K = 2
NHB = 2


def _moe_body(x_ref, rw_ref, gw_ref, uw_ref, dw_ref, out_ref):
    e = pl.program_id(0)
    hb = pl.program_id(1)
    N = x_ref.shape[0]

    CH = 256

    def step(i, _):
        sl = pl.ds(i * CH, CH)
        xc = x_ref[sl, :]

        logits = jnp.dot(xc, rw_ref[...], preferred_element_type=jnp.float32)
        lane = jax.lax.broadcasted_iota(jnp.int32, logits.shape, 1)
        l_e = jnp.sum(jnp.where(lane == e, logits, 0.0), axis=1, keepdims=True)
        cnt = jnp.sum(
            jnp.where((logits > l_e) | ((logits == l_e) & (lane < e)), 1, 0),
            axis=1, keepdims=True)
        m1 = jnp.max(logits, axis=1, keepdims=True)
        ft = jnp.min(jnp.where(logits == m1, lane, E), axis=1, keepdims=True)
        m2 = jnp.max(jnp.where(lane == ft, -jnp.inf, logits),
                     axis=1, keepdims=True)
        wc = jnp.where(cnt < K,
                       jnp.exp(l_e - m1) / (1.0 + jnp.exp(m2 - m1)), 0.0)

        g = jnp.dot(xc, gw_ref[0], preferred_element_type=jnp.float32)
        g = g * jax.nn.sigmoid(g)
        u = jnp.dot(xc, uw_ref[0], preferred_element_type=jnp.float32)
        eo = jnp.dot(g * u, dw_ref[0], preferred_element_type=jnp.float32)
        contrib = eo * wc

        @pl.when((e == 0) & (hb == 0))
        def _():
            out_ref[sl, :] = contrib

        @pl.when((e != 0) | (hb != 0))
        def _():
            out_ref[sl, :] += contrib

        return 0

    jax.lax.fori_loop(0, N // CH, step, 0)


def kernel(x, router_W, gate_W, up_W, down_W):
    B, S, D = x.shape
    H = gate_W.shape[2]
    HB = H // NHB
    xf = x.reshape(-1, D)
    N = xf.shape[0]

    out = pl.pallas_call(
        _moe_body,
        grid=(E, NHB),
        in_specs=[
            pl.BlockSpec((N, D), lambda e, hb: (0, 0)),
            pl.BlockSpec((D, E), lambda e, hb: (0, 0)),
            pl.BlockSpec((1, D, HB), lambda e, hb: (e, 0, hb)),
            pl.BlockSpec((1, D, HB), lambda e, hb: (e, 0, hb)),
            pl.BlockSpec((1, HB, D), lambda e, hb: (e, hb, 0)),
        ],
        out_specs=pl.BlockSpec((N, D), lambda e, hb: (0, 0)),
        out_shape=jax.ShapeDtypeStruct((N, D), jnp.float32),
    )(xf, router_W, gate_W, up_W, down_W)
    return out.reshape(B, S, D)


# speedup vs baseline: 1.9693x; 1.7407x over previous
import functools

import jax
import jax.numpy as jnp
from jax import lax
from jax.experimental import pallas as pl
from jax.experimental.pallas import tpu as pltpu
from jax.experimental.pallas import tpu_sc as plsc

E = 8
N = 4096
D = 1024
H = 2048
A = 2 * N
BM = 128
PAD = A + E * BM
NBLK = PAD // BM
NW = 32
APW = A // NW
TPW = N // NW
_NC = 2


def _router_body(x_ref, rw_ref, e_ref, w_ref, c0_ref, c1_ref):
    logits = jnp.dot(x_ref[...], rw_ref[...], preferred_element_type=jnp.float32)
    lane = lax.broadcasted_iota(jnp.int32, logits.shape, 1)
    m1 = jnp.max(logits, axis=1, keepdims=True)
    ft = jnp.min(jnp.where(logits == m1, lane, E), axis=1, keepdims=True)
    masked = jnp.where(lane == ft, -jnp.inf, logits)
    m2 = jnp.max(masked, axis=1, keepdims=True)
    st = jnp.min(jnp.where(masked == m2, lane, E), axis=1, keepdims=True)
    z = jnp.exp(m2 - m1)
    w1 = 1.0 / (1.0 + z)
    w2 = z / (1.0 + z)
    e_ref[...] = jnp.where(lane == 0, ft, jnp.where(lane == 1, st, 0))
    w_ref[...] = jnp.where(lane == 0, w1, jnp.where(lane == 1, w2, 0.0))
    lane16 = lax.broadcasted_iota(jnp.int32, (1, 16), 1)
    h0 = jnp.sum(jnp.where(lane == ft, 1, 0), axis=0, keepdims=True)
    h1 = jnp.sum(jnp.where(lane == st, 1, 0), axis=0, keepdims=True)
    zpad = jnp.zeros((1, 16 - E), jnp.int32)
    c0_ref[...] = jnp.concatenate([h0, zpad], axis=1)[None]
    c1_ref[...] = jnp.concatenate([h1, zpad], axis=1)[None]


def _router(xf, router_W):
    TB = 256
    return pl.pallas_call(
        _router_body,
        grid=(N // TB,),
        in_specs=[
            pl.BlockSpec((TB, D), lambda i: (i, 0)),
            pl.BlockSpec((D, E), lambda i: (0, 0)),
        ],
        out_specs=[
            pl.BlockSpec((TB, E), lambda i: (i, 0)),
            pl.BlockSpec((TB, E), lambda i: (i, 0)),
            pl.BlockSpec((1, 1, 16), lambda i: (i, 0, 0)),
            pl.BlockSpec((1, 1, 16), lambda i: (i, 0, 0)),
        ],
        out_shape=[
            jax.ShapeDtypeStruct((N, E), jnp.int32),
            jax.ShapeDtypeStruct((N, E), jnp.float32),
            jax.ShapeDtypeStruct((NW // 2, 1, 16), jnp.int32),
            jax.ShapeDtypeStruct((NW // 2, 1, 16), jnp.int32),
        ],
    )(xf, router_W)


def _sort_gather_body(eflat, c0_hbm, c1_hbm, x_hbm, xs_hbm, pos_hbm, bexp_hbm,
                      eids_v, pos_mat, tid_mat, ac0_v, ac1_v, xbufa, xbufb,
                      bexp_v, sh_v, gsema, gsemb, ssema, ssemb):
    wid = lax.axis_index("s") * _NC + lax.axis_index("c")
    iota16 = lax.iota(jnp.int32, 16)
    z16 = jnp.zeros((16,), jnp.int32)

    pltpu.sync_copy(eflat.at[pl.ds(wid * APW, APW)], eids_v)
    pltpu.sync_copy(c0_hbm, ac0_v)
    pltpu.sync_copy(c1_hbm, ac1_v)

    sh_v[pl.ds(0, 16)] = z16
    sh_v[pl.ds(32, 16)] = z16

    def shift_cumsum(t):
        for k in (1, 2, 4, 8):
            sh_v[pl.ds(16, 16)] = t
            t = t + sh_v[pl.ds(16 - k, 16)]
        return t

    def splat_lane(t, e):
        sh_v[pl.ds(16, 16)] = t
        a = sh_v[pl.ds(16 + e, 16)]
        u = jnp.where(iota16 == 0, a, 0)
        for k in (1, 2, 4, 8):
            sh_v[pl.ds(16, 16)] = u
            u = u + sh_v[pl.ds(16 - k, 16)]
        return u

    cnts = [z16] * E
    for s in range(APW // 16):
        v = eids_v[pl.ds(s * 16, 16)]
        for e in range(E):
            cnts[e] = cnts[e] + jnp.where(v == e, 1, 0)
    lanepref = [shift_cumsum(c) - c for c in cnts]

    tot = z16
    pre = z16
    for t in range(NW):
        row = ac0_v[t, 0] if t < NW // 2 else ac1_v[t - NW // 2, 0]
        tot = tot + row
        flag = ((jnp.int32(t) - wid) >> 31) & 1
        pre = pre + row * flag

    rt = ((tot + BM - 1) >> 7) << 7
    base = shift_cumsum(rt) - rt
    startv = base + pre

    runs = []
    for e in range(E):
        runs.append(splat_lane(startv, e) + lanepref[e])

    for s in range(APW // 16):
        v = eids_v[pl.ds(s * 16, 16)]
        pos = z16
        for e in range(E):
            m = v == e
            pos = pos + jnp.where(m, runs[e], 0)
            runs[e] = runs[e] + jnp.where(m, 1, 0)
        pos_mat[s // 2, pl.ds((s % 2) * 16, 16)] = pos
        tid_mat[s // 2, pl.ds((s % 2) * 16, 16)] = (
            (wid * APW + s * 16 + iota16) & (N - 1))

    pltpu.sync_copy(pos_mat, pos_hbm.at[pl.ds(wid * 8, 8)])

    bufs = (xbufa, xbufb)
    gsems = (gsema, gsemb)
    ssems = (ssema, ssemb)
    NCH = APW // 32
    g = {}
    sc = {}
    g[0] = pltpu.async_copy(x_hbm.at[tid_mat.at[0]], bufs[0], gsems[0])
    for c in range(NCH):
        if c + 1 < NCH:
            if c >= 1:
                sc[c - 1].wait()
            g[c + 1] = pltpu.async_copy(
                x_hbm.at[tid_mat.at[c + 1]], bufs[(c + 1) % 2],
                gsems[(c + 1) % 2])
        g[c].wait()
        sc[c] = pltpu.async_copy(
            bufs[c % 2], xs_hbm.at[pos_mat.at[c]], ssems[c % 2])
    sc[NCH - 2].wait()
    sc[NCH - 1].wait()

    @pl.when(wid == 0)
    def _():
        bes = [splat_lane(base, e) for e in range(1, E)]
        for j in range(5):
            bs = (j * 16 + iota16) << 7
            expv = z16
            for be in bes:
                expv = expv + jnp.where(bs >= be, 1, 0)
            bexp_v[pl.ds(j * 16, 16)] = expv
        pltpu.sync_copy(bexp_v, bexp_hbm)


def _sort_gather(*args):
    return functools.partial(
        pl.kernel,
        out_type=[
            jax.ShapeDtypeStruct((PAD, D), jnp.float32),
            jax.ShapeDtypeStruct((A // 32, 32), jnp.int32),
            jax.ShapeDtypeStruct((80,), jnp.int32),
        ],
        mesh=plsc.VectorSubcoreMesh(core_axis_name="c", subcore_axis_name="s"),
        scratch_types=[
            pltpu.VMEM((APW,), jnp.int32),
            pltpu.VMEM((8, 32), jnp.int32),
            pltpu.VMEM((8, 32), jnp.int32),
            pltpu.VMEM((NW // 2, 1, 16), jnp.int32),
            pltpu.VMEM((NW // 2, 1, 16), jnp.int32),
            pltpu.VMEM((32, D), jnp.float32),
            pltpu.VMEM((32, D), jnp.float32),
            pltpu.VMEM((80,), jnp.int32),
            pltpu.VMEM((48,), jnp.int32),
            pltpu.SemaphoreType.DMA,
            pltpu.SemaphoreType.DMA,
            pltpu.SemaphoreType.DMA,
            pltpu.SemaphoreType.DMA,
        ],
    )(_sort_gather_body)(*args)


def _mlp_body(be_ref, xs_ref, gw_ref, uw_ref, dw_ref, ys_ref):
    xc = xs_ref[...]
    g = jnp.dot(xc, gw_ref[0], preferred_element_type=jnp.float32)
    g = g * jax.nn.sigmoid(g)
    u = jnp.dot(xc, uw_ref[0], preferred_element_type=jnp.float32)
    ys_ref[...] = jnp.dot(g * u, dw_ref[0], preferred_element_type=jnp.float32)


def _mlp(bexp, xs, gate_W, up_W, down_W):
    grid_spec = pltpu.PrefetchScalarGridSpec(
        num_scalar_prefetch=1,
        grid=(NBLK,),
        in_specs=[
            pl.BlockSpec((BM, D), lambda b, be: (b, 0)),
            pl.BlockSpec((1, D, H), lambda b, be: (be[b], 0, 0)),
            pl.BlockSpec((1, D, H), lambda b, be: (be[b], 0, 0)),
            pl.BlockSpec((1, H, D), lambda b, be: (be[b], 0, 0)),
        ],
        out_specs=pl.BlockSpec((BM, D), lambda b, be: (b, 0)),
    )
    return pl.pallas_call(
        _mlp_body,
        grid_spec=grid_spec,
        out_shape=jax.ShapeDtypeStruct((PAD, D), jnp.float32),
    )(bexp, xs, gate_W, up_W, down_W)


def _pair_body(ys_hbm, pos_hbm, z0_hbm, z1_hbm,
               p0_mat, p1_mat, yba, ybb, gsema, gsemb, wsema, wsemb):
    wid = lax.axis_index("s") * _NC + lax.axis_index("c")
    pltpu.sync_copy(pos_hbm.at[pl.ds(4 * wid, 4)], p0_mat)
    pltpu.sync_copy(pos_hbm.at[pl.ds(128 + 4 * wid, 4)], p1_mat)

    bufs = (yba, ybb)
    gsems = (gsema, gsemb)
    wsems = (wsema, wsemb)
    t0 = wid * TPW

    def src(t):
        k, c = divmod(t, 4)
        m = p0_mat if k == 0 else p1_mat
        return m.at[c]

    def dst(t):
        k, c = divmod(t, 4)
        z = z0_hbm if k == 0 else z1_hbm
        return z.at[pl.ds(t0 + c * 32, 32)]

    g = {}
    w = {}
    g[0] = pltpu.async_copy(ys_hbm.at[src(0)], bufs[0], gsems[0])
    for t in range(8):
        if t + 1 < 8:
            if t >= 1:
                w[t - 1].wait()
            g[t + 1] = pltpu.async_copy(
                ys_hbm.at[src(t + 1)], bufs[(t + 1) % 2], gsems[(t + 1) % 2])
        g[t].wait()
        w[t] = pltpu.async_copy(bufs[t % 2], dst(t), wsems[t % 2])
    w[6].wait()
    w[7].wait()


def _pair_gather(*args):
    return functools.partial(
        pl.kernel,
        out_type=[
            jax.ShapeDtypeStruct((N, D), jnp.float32),
            jax.ShapeDtypeStruct((N, D), jnp.float32),
        ],
        mesh=plsc.VectorSubcoreMesh(core_axis_name="c", subcore_axis_name="s"),
        scratch_types=[
            pltpu.VMEM((4, 32), jnp.int32),
            pltpu.VMEM((4, 32), jnp.int32),
            pltpu.VMEM((32, D), jnp.float32),
            pltpu.VMEM((32, D), jnp.float32),
            pltpu.SemaphoreType.DMA,
            pltpu.SemaphoreType.DMA,
            pltpu.SemaphoreType.DMA,
            pltpu.SemaphoreType.DMA,
        ],
    )(_pair_body)(*args)


def _comb_body(z0_ref, z1_ref, w_ref, out_ref):
    wb = w_ref[...]
    out_ref[...] = z0_ref[...] * wb[:, 0:1] + z1_ref[...] * wb[:, 1:2]


def _comb(z0, z1, w8):
    TB = 512
    return pl.pallas_call(
        _comb_body,
        grid=(N // TB,),
        in_specs=[
            pl.BlockSpec((TB, D), lambda i: (i, 0)),
            pl.BlockSpec((TB, D), lambda i: (i, 0)),
            pl.BlockSpec((TB, E), lambda i: (i, 0)),
        ],
        out_specs=pl.BlockSpec((TB, D), lambda i: (i, 0)),
        out_shape=jax.ShapeDtypeStruct((N, D), jnp.float32),
    )(z0, z1, w8)


def kernel(x, router_W, gate_W, up_W, down_W):
    B, S, _ = x.shape
    xf = x.reshape(-1, D)
    e8, w8, c0, c1 = _router(xf, router_W)
    eflat = jnp.concatenate([e8[:, 0], e8[:, 1]])
    xs, pos2d, bexp = _sort_gather(eflat, c0, c1, xf)
    ys = _mlp(bexp[:NBLK], xs, gate_W, up_W, down_W)
    z0, z1 = _pair_gather(ys, pos2d)
    out = _comb(z0, z1, w8)
    return out.reshape(B, S, D)


# speedup vs baseline: 2.0397x; 1.0357x over previous
import functools

import jax
import jax.numpy as jnp
from jax import lax
from jax.experimental import pallas as pl
from jax.experimental.pallas import tpu as pltpu
from jax.experimental.pallas import tpu_sc as plsc

E = 8
N = 4096
D = 1024
H = 2048
A = 2 * N
BM = 128
PAD = A + E * BM
NBLK = PAD // BM
NW = 32
APW = A // NW
TPW = N // NW
_NC = 2


def _router_body(x_ref, rw_ref, e_ref, w_ref, c0_ref, c1_ref):
    logits = jnp.dot(x_ref[...], rw_ref[...], preferred_element_type=jnp.float32)
    lane = lax.broadcasted_iota(jnp.int32, logits.shape, 1)
    m1 = jnp.max(logits, axis=1, keepdims=True)
    ft = jnp.min(jnp.where(logits == m1, lane, E), axis=1, keepdims=True)
    masked = jnp.where(lane == ft, -jnp.inf, logits)
    m2 = jnp.max(masked, axis=1, keepdims=True)
    st = jnp.min(jnp.where(masked == m2, lane, E), axis=1, keepdims=True)
    z = jnp.exp(m2 - m1)
    w1 = 1.0 / (1.0 + z)
    w2 = z / (1.0 + z)
    e_ref[...] = jnp.where(lane == 0, ft, jnp.where(lane == 1, st, 0))
    w_ref[...] = jnp.where(lane == 0, w1, jnp.where(lane == 1, w2, 0.0))
    lane16 = lax.broadcasted_iota(jnp.int32, (1, 16), 1)
    h0 = jnp.sum(jnp.where(lane == ft, 1, 0), axis=0, keepdims=True)
    h1 = jnp.sum(jnp.where(lane == st, 1, 0), axis=0, keepdims=True)
    zpad = jnp.zeros((1, 16 - E), jnp.int32)
    c0_ref[...] = jnp.concatenate([h0, zpad], axis=1)[None]
    c1_ref[...] = jnp.concatenate([h1, zpad], axis=1)[None]


def _router(xf, router_W):
    TB = 256
    return pl.pallas_call(
        _router_body,
        grid=(N // TB,),
        in_specs=[
            pl.BlockSpec((TB, D), lambda i: (i, 0)),
            pl.BlockSpec((D, E), lambda i: (0, 0)),
        ],
        out_specs=[
            pl.BlockSpec((TB, E), lambda i: (i, 0)),
            pl.BlockSpec((TB, E), lambda i: (i, 0)),
            pl.BlockSpec((1, 1, 16), lambda i: (i, 0, 0)),
            pl.BlockSpec((1, 1, 16), lambda i: (i, 0, 0)),
        ],
        out_shape=[
            jax.ShapeDtypeStruct((N, E), jnp.int32),
            jax.ShapeDtypeStruct((N, E), jnp.float32),
            jax.ShapeDtypeStruct((NW // 2, 1, 16), jnp.int32),
            jax.ShapeDtypeStruct((NW // 2, 1, 16), jnp.int32),
        ],
    )(xf, router_W)


def _sort_gather_body(eflat, c0_hbm, c1_hbm, x_hbm, xs_hbm, pos_hbm, bexp_hbm,
                      eids_v, pos_mat, tid_mat, ac0_v, ac1_v, xbufa, xbufb,
                      bexp_v, sh_v, gsema, gsemb, ssema, ssemb):
    wid = lax.axis_index("s") * _NC + lax.axis_index("c")
    iota16 = lax.iota(jnp.int32, 16)
    z16 = jnp.zeros((16,), jnp.int32)

    pltpu.sync_copy(eflat.at[pl.ds(wid * APW, APW)], eids_v)
    pltpu.sync_copy(c0_hbm, ac0_v)
    pltpu.sync_copy(c1_hbm, ac1_v)

    sh_v[pl.ds(0, 16)] = z16
    sh_v[pl.ds(32, 16)] = z16

    def shift_cumsum(t):
        for k in (1, 2, 4, 8):
            sh_v[pl.ds(16, 16)] = t
            t = t + sh_v[pl.ds(16 - k, 16)]
        return t

    def splat_lane(t, e):
        sh_v[pl.ds(16, 16)] = t
        a = sh_v[pl.ds(16 + e, 16)]
        u = jnp.where(iota16 == 0, a, 0)
        for k in (1, 2, 4, 8):
            sh_v[pl.ds(16, 16)] = u
            u = u + sh_v[pl.ds(16 - k, 16)]
        return u

    cnts = [z16] * E
    for s in range(APW // 16):
        v = eids_v[pl.ds(s * 16, 16)]
        for e in range(E):
            cnts[e] = cnts[e] + jnp.where(v == e, 1, 0)
    lanepref = [shift_cumsum(c) - c for c in cnts]

    tot = z16
    pre = z16
    for t in range(NW):
        row = ac0_v[t, 0] if t < NW // 2 else ac1_v[t - NW // 2, 0]
        tot = tot + row
        flag = ((jnp.int32(t) - wid) >> 31) & 1
        pre = pre + row * flag

    rt = ((tot + BM - 1) >> 7) << 7
    base = shift_cumsum(rt) - rt
    startv = base + pre

    runs = []
    for e in range(E):
        runs.append(splat_lane(startv, e) + lanepref[e])

    for s in range(APW // 16):
        v = eids_v[pl.ds(s * 16, 16)]
        pos = z16
        for e in range(E):
            m = v == e
            pos = pos + jnp.where(m, runs[e], 0)
            runs[e] = runs[e] + jnp.where(m, 1, 0)
        pos_mat[s // 2, pl.ds((s % 2) * 16, 16)] = pos
        tid_mat[s // 2, pl.ds((s % 2) * 16, 16)] = (
            (wid * APW + s * 16 + iota16) & (N - 1))

    pltpu.sync_copy(pos_mat, pos_hbm.at[pl.ds(wid * 8, 8)])

    bufs = (xbufa, xbufb)
    gsems = (gsema, gsemb)
    ssems = (ssema, ssemb)
    NCH = APW // 32
    g = {}
    sc = {}
    g[0] = pltpu.async_copy(x_hbm.at[tid_mat.at[0]], bufs[0], gsems[0])
    for c in range(NCH):
        if c + 1 < NCH:
            if c >= 1:
                sc[c - 1].wait()
            g[c + 1] = pltpu.async_copy(
                x_hbm.at[tid_mat.at[c + 1]], bufs[(c + 1) % 2],
                gsems[(c + 1) % 2])
        g[c].wait()
        sc[c] = pltpu.async_copy(
            bufs[c % 2], xs_hbm.at[pos_mat.at[c]], ssems[c % 2])
    sc[NCH - 2].wait()
    sc[NCH - 1].wait()

    @pl.when(wid == 0)
    def _():
        bes = [splat_lane(base, e) for e in range(1, E)]
        for j in range(5):
            bs = (j * 16 + iota16) << 7
            expv = z16
            for be in bes:
                expv = expv + jnp.where(bs >= be, 1, 0)
            bexp_v[pl.ds(j * 16, 16)] = expv
        pltpu.sync_copy(bexp_v, bexp_hbm)


def _sort_gather(*args):
    return functools.partial(
        pl.kernel,
        out_type=[
            jax.ShapeDtypeStruct((PAD, D), jnp.float32),
            jax.ShapeDtypeStruct((A // 32, 32), jnp.int32),
            jax.ShapeDtypeStruct((80,), jnp.int32),
        ],
        mesh=plsc.VectorSubcoreMesh(core_axis_name="c", subcore_axis_name="s"),
        scratch_types=[
            pltpu.VMEM((APW,), jnp.int32),
            pltpu.VMEM((8, 32), jnp.int32),
            pltpu.VMEM((8, 32), jnp.int32),
            pltpu.VMEM((NW // 2, 1, 16), jnp.int32),
            pltpu.VMEM((NW // 2, 1, 16), jnp.int32),
            pltpu.VMEM((32, D), jnp.float32),
            pltpu.VMEM((32, D), jnp.float32),
            pltpu.VMEM((80,), jnp.int32),
            pltpu.VMEM((48,), jnp.int32),
            pltpu.SemaphoreType.DMA,
            pltpu.SemaphoreType.DMA,
            pltpu.SemaphoreType.DMA,
            pltpu.SemaphoreType.DMA,
        ],
    )(_sort_gather_body)(*args)


def _mlp_body(be_ref, xs_ref, gw_ref, uw_ref, dw_ref, ys_ref):
    xc = xs_ref[...]
    g = jnp.dot(xc, gw_ref[0], preferred_element_type=jnp.float32)
    g = g * jax.nn.sigmoid(g)
    u = jnp.dot(xc, uw_ref[0], preferred_element_type=jnp.float32)
    ys_ref[...] = jnp.dot(g * u, dw_ref[0], preferred_element_type=jnp.float32)


def _mlp(bexp, xs, gate_W, up_W, down_W):
    grid_spec = pltpu.PrefetchScalarGridSpec(
        num_scalar_prefetch=1,
        grid=(NBLK,),
        in_specs=[
            pl.BlockSpec((BM, D), lambda b, be: (b, 0)),
            pl.BlockSpec((1, D, H), lambda b, be: (be[b], 0, 0)),
            pl.BlockSpec((1, D, H), lambda b, be: (be[b], 0, 0)),
            pl.BlockSpec((1, H, D), lambda b, be: (be[b], 0, 0)),
        ],
        out_specs=pl.BlockSpec((BM, D), lambda b, be: (b, 0)),
    )
    return pl.pallas_call(
        _mlp_body,
        grid_spec=grid_spec,
        out_shape=jax.ShapeDtypeStruct((PAD, D), jnp.float32),
    )(bexp, xs, gate_W, up_W, down_W)


def _combine_body(ys_hbm, pos_hbm, w0_hbm, w1_hbm, out_hbm,
                  p0_mat, p1_mat, w0_v, w1_v, shf_v,
                  yb0a, yb1a, oba, yb0b, yb1b, obb,
                  g0a, g1a, wsa, g0b, g1b, wsb):
    wid = lax.axis_index("s") * _NC + lax.axis_index("c")
    iota16 = lax.iota(jnp.int32, 16)
    zf16 = jnp.zeros((16,), jnp.float32)
    t0 = wid * TPW

    pltpu.sync_copy(pos_hbm.at[pl.ds(4 * wid, 4)], p0_mat)
    pltpu.sync_copy(pos_hbm.at[pl.ds(128 + 4 * wid, 4)], p1_mat)
    pltpu.sync_copy(w0_hbm.at[pl.ds(t0, TPW)], w0_v.at[pl.ds(0, TPW)])
    pltpu.sync_copy(w1_hbm.at[pl.ds(t0, TPW)], w1_v.at[pl.ds(0, TPW)])
    w0_v[pl.ds(TPW, 16)] = zf16
    w1_v[pl.ds(TPW, 16)] = zf16
    shf_v[pl.ds(0, 16)] = zf16
    shf_v[pl.ds(32, 16)] = zf16

    sets = ((yb0a, yb1a, oba, g0a, g1a, wsa),
            (yb0b, yb1b, obb, g0b, g1b, wsb))

    def splat(ref, ri):
        a = ref[pl.ds(ri, 16)]
        u = jnp.where(iota16 == 0, a, 0.0)
        for k in (1, 2, 4, 8):
            shf_v[pl.ds(16, 16)] = u
            u = u + shf_v[pl.ds(16 - k, 16)]
        return u

    NCH = TPW // 16

    def idx0(c):
        return p0_mat.at[c // 2, pl.ds((c % 2) * 16, 16)]

    def idx1(c):
        return p1_mat.at[c // 2, pl.ds((c % 2) * 16, 16)]

    h = {}
    yb0, yb1, ob, sg0, sg1, sw = sets[0]
    h[(0, 0)] = pltpu.async_copy(ys_hbm.at[idx0(0)], yb0, sg0)
    h[(0, 1)] = pltpu.async_copy(ys_hbm.at[idx1(0)], yb1, sg1)
    for c in range(NCH):
        yb0, yb1, ob, sg0, sg1, sw = sets[c % 2]
        if c + 1 < NCH:
            nb0, nb1, nob, ng0, ng1, nw = sets[(c + 1) % 2]
            if c >= 1:
                h[(c - 1, "w")].wait()
            h[(c + 1, 0)] = pltpu.async_copy(ys_hbm.at[idx0(c + 1)], nb0, ng0)
            h[(c + 1, 1)] = pltpu.async_copy(ys_hbm.at[idx1(c + 1)], nb1, ng1)
        h[(c, 0)].wait()
        h[(c, 1)].wait()

        def row_body(r, _, yb0=yb0, yb1=yb1, ob=ob, c=c):
            ri = c * 16 + r
            w0s = splat(w0_v, ri)
            w1s = splat(w1_v, ri)
            for q in range(D // 16):
                sl = pl.ds(q * 16, 16)
                ob[r, sl] = yb0[r, sl] * w0s + yb1[r, sl] * w1s
            return 0

        lax.fori_loop(0, 16, row_body, 0)
        h[(c, "w")] = pltpu.async_copy(
            ob, out_hbm.at[pl.ds(t0 + c * 16, 16)], sw)
    h[(NCH - 2, "w")].wait()
    h[(NCH - 1, "w")].wait()


def _combine(*args):
    return functools.partial(
        pl.kernel,
        out_type=jax.ShapeDtypeStruct((N, D), jnp.float32),
        mesh=plsc.VectorSubcoreMesh(core_axis_name="c", subcore_axis_name="s"),
        scratch_types=[
            pltpu.VMEM((4, 32), jnp.int32),
            pltpu.VMEM((4, 32), jnp.int32),
            pltpu.VMEM((TPW + 16,), jnp.float32),
            pltpu.VMEM((TPW + 16,), jnp.float32),
            pltpu.VMEM((48,), jnp.float32),
            pltpu.VMEM((16, D), jnp.float32),
            pltpu.VMEM((16, D), jnp.float32),
            pltpu.VMEM((16, D), jnp.float32),
            pltpu.VMEM((16, D), jnp.float32),
            pltpu.VMEM((16, D), jnp.float32),
            pltpu.VMEM((16, D), jnp.float32),
            pltpu.SemaphoreType.DMA,
            pltpu.SemaphoreType.DMA,
            pltpu.SemaphoreType.DMA,
            pltpu.SemaphoreType.DMA,
            pltpu.SemaphoreType.DMA,
            pltpu.SemaphoreType.DMA,
        ],
    )(_combine_body)(*args)


def kernel(x, router_W, gate_W, up_W, down_W):
    B, S, _ = x.shape
    xf = x.reshape(-1, D)
    e8, w8, c0, c1 = _router(xf, router_W)
    eflat = jnp.concatenate([e8[:, 0], e8[:, 1]])
    xs, pos2d, bexp = _sort_gather(eflat, c0, c1, xf)
    ys = _mlp(bexp[:NBLK], xs, gate_W, up_W, down_W)
    out = _combine(ys, pos2d, w8[:, 0], w8[:, 1])
    return out.reshape(B, S, D)


# speedup vs baseline: 2.0411x; 1.0007x over previous
import functools

import jax
import jax.numpy as jnp
from jax import lax
from jax.experimental import pallas as pl
from jax.experimental.pallas import tpu as pltpu
from jax.experimental.pallas import tpu_sc as plsc

E = 8
N = 4096
D = 1024
H = 2048
A = 2 * N
BM = 128
PAD = A + E * BM
NBLK = PAD // BM
NW = 32
APW = A // NW
TPW = N // NW
_NC = 2


def _router_body(x_ref, rw_ref, e_ref, w_ref, c0_ref, c1_ref):
    logits = jnp.dot(x_ref[...], rw_ref[...], preferred_element_type=jnp.float32)
    lane = lax.broadcasted_iota(jnp.int32, logits.shape, 1)
    m1 = jnp.max(logits, axis=1, keepdims=True)
    ft = jnp.min(jnp.where(logits == m1, lane, E), axis=1, keepdims=True)
    masked = jnp.where(lane == ft, -jnp.inf, logits)
    m2 = jnp.max(masked, axis=1, keepdims=True)
    st = jnp.min(jnp.where(masked == m2, lane, E), axis=1, keepdims=True)
    z = jnp.exp(m2 - m1)
    w1 = 1.0 / (1.0 + z)
    w2 = z / (1.0 + z)
    e_ref[...] = jnp.where(lane == 0, ft, jnp.where(lane == 1, st, 0))
    w_ref[...] = jnp.where(lane == 0, w1, jnp.where(lane == 1, w2, 0.0))
    lane16 = lax.broadcasted_iota(jnp.int32, (1, 16), 1)
    h0 = jnp.sum(jnp.where(lane == ft, 1, 0), axis=0, keepdims=True)
    h1 = jnp.sum(jnp.where(lane == st, 1, 0), axis=0, keepdims=True)
    zpad = jnp.zeros((1, 16 - E), jnp.int32)
    c0_ref[...] = jnp.concatenate([h0, zpad], axis=1)[None]
    c1_ref[...] = jnp.concatenate([h1, zpad], axis=1)[None]


def _router(xf, router_W):
    TB = 256
    return pl.pallas_call(
        _router_body,
        grid=(N // TB,),
        in_specs=[
            pl.BlockSpec((TB, D), lambda i: (i, 0)),
            pl.BlockSpec((D, E), lambda i: (0, 0)),
        ],
        out_specs=[
            pl.BlockSpec((TB, E), lambda i: (i, 0)),
            pl.BlockSpec((TB, E), lambda i: (i, 0)),
            pl.BlockSpec((1, 1, 16), lambda i: (i, 0, 0)),
            pl.BlockSpec((1, 1, 16), lambda i: (i, 0, 0)),
        ],
        out_shape=[
            jax.ShapeDtypeStruct((N, E), jnp.int32),
            jax.ShapeDtypeStruct((N, E), jnp.float32),
            jax.ShapeDtypeStruct((NW // 2, 1, 16), jnp.int32),
            jax.ShapeDtypeStruct((NW // 2, 1, 16), jnp.int32),
        ],
    )(xf, router_W)


def _sort_gather_body(eflat, c0_hbm, c1_hbm, x_hbm, xs_hbm, pos_hbm, bexp_hbm,
                      eids_v, pos_mat, ac0_v, ac1_v, xbufa, xbufb,
                      bexp_v, sh_v, gsema, gsemb, ssema, ssemb):
    wid = lax.axis_index("s") * _NC + lax.axis_index("c")
    iota16 = lax.iota(jnp.int32, 16)
    z16 = jnp.zeros((16,), jnp.int32)

    pltpu.sync_copy(eflat.at[pl.ds(wid * APW, APW)], eids_v)
    pltpu.sync_copy(c0_hbm, ac0_v)
    pltpu.sync_copy(c1_hbm, ac1_v)

    sh_v[pl.ds(0, 16)] = z16
    sh_v[pl.ds(32, 16)] = z16

    def shift_cumsum(t):
        for k in (1, 2, 4, 8):
            sh_v[pl.ds(16, 16)] = t
            t = t + sh_v[pl.ds(16 - k, 16)]
        return t

    def splat_lane(t, e):
        sh_v[pl.ds(16, 16)] = t
        a = sh_v[pl.ds(16 + e, 16)]
        u = jnp.where(iota16 == 0, a, 0)
        for k in (1, 2, 4, 8):
            sh_v[pl.ds(16, 16)] = u
            u = u + sh_v[pl.ds(16 - k, 16)]
        return u

    cnts = [z16] * E
    for s in range(APW // 16):
        v = eids_v[pl.ds(s * 16, 16)]
        for e in range(E):
            cnts[e] = cnts[e] + jnp.where(v == e, 1, 0)
    lanepref = [shift_cumsum(c) - c for c in cnts]

    tot = z16
    pre = z16
    for t in range(NW):
        row = ac0_v[t, 0] if t < NW // 2 else ac1_v[t - NW // 2, 0]
        tot = tot + row
        flag = ((jnp.int32(t) - wid) >> 31) & 1
        pre = pre + row * flag

    rt = ((tot + BM - 1) >> 7) << 7
    base = shift_cumsum(rt) - rt
    startv = base + pre

    runs = []
    for e in range(E):
        runs.append(splat_lane(startv, e) + lanepref[e])

    for s in range(APW // 16):
        v = eids_v[pl.ds(s * 16, 16)]
        pos = z16
        for e in range(E):
            m = v == e
            pos = pos + jnp.where(m, runs[e], 0)
            runs[e] = runs[e] + jnp.where(m, 1, 0)
        pos_mat[s // 2, pl.ds((s % 2) * 16, 16)] = pos

    pltpu.sync_copy(pos_mat, pos_hbm.at[pl.ds(wid * 8, 8)])

    bufs = (xbufa, xbufb)
    gsems = (gsema, gsemb)
    ssems = (ssema, ssemb)
    NCH = APW // 32

    def xsrc(c):
        off = pl.multiple_of((wid * APW + c * 32) & (N - 1), 32)
        return x_hbm.at[pl.ds(off, 32)]

    g = {}
    sc = {}
    g[0] = pltpu.async_copy(xsrc(0), bufs[0], gsems[0])
    for c in range(NCH):
        if c + 1 < NCH:
            if c >= 1:
                sc[c - 1].wait()
            g[c + 1] = pltpu.async_copy(xsrc(c + 1), bufs[(c + 1) % 2],
                                        gsems[(c + 1) % 2])
        g[c].wait()
        sc[c] = pltpu.async_copy(
            bufs[c % 2], xs_hbm.at[pos_mat.at[c]], ssems[c % 2])
    sc[NCH - 2].wait()
    sc[NCH - 1].wait()

    @pl.when(wid == 0)
    def _():
        bes = [splat_lane(base, e) for e in range(1, E)]
        for j in range(5):
            bs = (j * 16 + iota16) << 7
            expv = z16
            for be in bes:
                expv = expv + jnp.where(bs >= be, 1, 0)
            bexp_v[pl.ds(j * 16, 16)] = expv
        pltpu.sync_copy(bexp_v, bexp_hbm)


def _sort_gather(*args):
    return functools.partial(
        pl.kernel,
        out_type=[
            jax.ShapeDtypeStruct((PAD, D), jnp.float32),
            jax.ShapeDtypeStruct((A // 32, 32), jnp.int32),
            jax.ShapeDtypeStruct((80,), jnp.int32),
        ],
        mesh=plsc.VectorSubcoreMesh(core_axis_name="c", subcore_axis_name="s"),
        scratch_types=[
            pltpu.VMEM((APW,), jnp.int32),
            pltpu.VMEM((8, 32), jnp.int32),
            pltpu.VMEM((NW // 2, 1, 16), jnp.int32),
            pltpu.VMEM((NW // 2, 1, 16), jnp.int32),
            pltpu.VMEM((32, D), jnp.float32),
            pltpu.VMEM((32, D), jnp.float32),
            pltpu.VMEM((80,), jnp.int32),
            pltpu.VMEM((48,), jnp.int32),
            pltpu.SemaphoreType.DMA,
            pltpu.SemaphoreType.DMA,
            pltpu.SemaphoreType.DMA,
            pltpu.SemaphoreType.DMA,
        ],
    )(_sort_gather_body)(*args)


def _mlp_body(be_ref, xs_ref, gw_ref, uw_ref, dw_ref, ys_ref):
    xc = xs_ref[...]
    g = jnp.dot(xc, gw_ref[0], preferred_element_type=jnp.float32)
    g = g * jax.nn.sigmoid(g)
    u = jnp.dot(xc, uw_ref[0], preferred_element_type=jnp.float32)
    ys_ref[...] = jnp.dot(g * u, dw_ref[0], preferred_element_type=jnp.float32)


def _mlp(bexp, xs, gate_W, up_W, down_W):
    grid_spec = pltpu.PrefetchScalarGridSpec(
        num_scalar_prefetch=1,
        grid=(NBLK,),
        in_specs=[
            pl.BlockSpec((BM, D), lambda b, be: (b, 0)),
            pl.BlockSpec((1, D, H), lambda b, be: (be[b], 0, 0)),
            pl.BlockSpec((1, D, H), lambda b, be: (be[b], 0, 0)),
            pl.BlockSpec((1, H, D), lambda b, be: (be[b], 0, 0)),
        ],
        out_specs=pl.BlockSpec((BM, D), lambda b, be: (b, 0)),
    )
    return pl.pallas_call(
        _mlp_body,
        grid_spec=grid_spec,
        out_shape=jax.ShapeDtypeStruct((PAD, D), jnp.float32),
    )(bexp, xs, gate_W, up_W, down_W)


def _combine_body(ys_hbm, pos_hbm, w0_hbm, w1_hbm, out_hbm,
                  p0_mat, p1_mat, w0_v, w1_v, shf_v,
                  yb0a, yb1a, oba, yb0b, yb1b, obb,
                  g0a, g1a, wsa, g0b, g1b, wsb):
    wid = lax.axis_index("s") * _NC + lax.axis_index("c")
    iota16 = lax.iota(jnp.int32, 16)
    zf16 = jnp.zeros((16,), jnp.float32)
    t0 = wid * TPW

    pltpu.sync_copy(pos_hbm.at[pl.ds(4 * wid, 4)], p0_mat)
    pltpu.sync_copy(pos_hbm.at[pl.ds(128 + 4 * wid, 4)], p1_mat)
    pltpu.sync_copy(w0_hbm.at[pl.ds(t0, TPW)], w0_v.at[pl.ds(0, TPW)])
    pltpu.sync_copy(w1_hbm.at[pl.ds(t0, TPW)], w1_v.at[pl.ds(0, TPW)])
    w0_v[pl.ds(TPW, 16)] = zf16
    w1_v[pl.ds(TPW, 16)] = zf16
    shf_v[pl.ds(0, 16)] = zf16
    shf_v[pl.ds(32, 16)] = zf16

    sets = ((yb0a, yb1a, oba, g0a, g1a, wsa),
            (yb0b, yb1b, obb, g0b, g1b, wsb))

    def splat(ref, ri):
        a = ref[pl.ds(ri, 16)]
        u = jnp.where(iota16 == 0, a, 0.0)
        for k in (1, 2, 4, 8):
            shf_v[pl.ds(16, 16)] = u
            u = u + shf_v[pl.ds(16 - k, 16)]
        return u

    NCH = TPW // 16

    def idx0(c):
        return p0_mat.at[c // 2, pl.ds((c % 2) * 16, 16)]

    def idx1(c):
        return p1_mat.at[c // 2, pl.ds((c % 2) * 16, 16)]

    h = {}
    yb0, yb1, ob, sg0, sg1, sw = sets[0]
    h[(0, 0)] = pltpu.async_copy(ys_hbm.at[idx0(0)], yb0, sg0)
    h[(0, 1)] = pltpu.async_copy(ys_hbm.at[idx1(0)], yb1, sg1)
    for c in range(NCH):
        yb0, yb1, ob, sg0, sg1, sw = sets[c % 2]
        if c + 1 < NCH:
            nb0, nb1, nob, ng0, ng1, nw = sets[(c + 1) % 2]
            if c >= 1:
                h[(c - 1, "w")].wait()
            h[(c + 1, 0)] = pltpu.async_copy(ys_hbm.at[idx0(c + 1)], nb0, ng0)
            h[(c + 1, 1)] = pltpu.async_copy(ys_hbm.at[idx1(c + 1)], nb1, ng1)
        h[(c, 0)].wait()
        h[(c, 1)].wait()

        def row_body(r, _, yb0=yb0, yb1=yb1, ob=ob, c=c):
            ri = c * 16 + r
            w0s = splat(w0_v, ri)
            w1s = splat(w1_v, ri)
            for q in range(D // 16):
                sl = pl.ds(q * 16, 16)
                ob[r, sl] = yb0[r, sl] * w0s + yb1[r, sl] * w1s
            return 0

        lax.fori_loop(0, 16, row_body, 0)
        h[(c, "w")] = pltpu.async_copy(
            ob, out_hbm.at[pl.ds(t0 + c * 16, 16)], sw)
    h[(NCH - 2, "w")].wait()
    h[(NCH - 1, "w")].wait()


def _combine(*args):
    return functools.partial(
        pl.kernel,
        out_type=jax.ShapeDtypeStruct((N, D), jnp.float32),
        mesh=plsc.VectorSubcoreMesh(core_axis_name="c", subcore_axis_name="s"),
        scratch_types=[
            pltpu.VMEM((4, 32), jnp.int32),
            pltpu.VMEM((4, 32), jnp.int32),
            pltpu.VMEM((TPW + 16,), jnp.float32),
            pltpu.VMEM((TPW + 16,), jnp.float32),
            pltpu.VMEM((48,), jnp.float32),
            pltpu.VMEM((16, D), jnp.float32),
            pltpu.VMEM((16, D), jnp.float32),
            pltpu.VMEM((16, D), jnp.float32),
            pltpu.VMEM((16, D), jnp.float32),
            pltpu.VMEM((16, D), jnp.float32),
            pltpu.VMEM((16, D), jnp.float32),
            pltpu.SemaphoreType.DMA,
            pltpu.SemaphoreType.DMA,
            pltpu.SemaphoreType.DMA,
            pltpu.SemaphoreType.DMA,
            pltpu.SemaphoreType.DMA,
            pltpu.SemaphoreType.DMA,
        ],
    )(_combine_body)(*args)


def kernel(x, router_W, gate_W, up_W, down_W):
    B, S, _ = x.shape
    xf = x.reshape(-1, D)
    e8, w8, c0, c1 = _router(xf, router_W)
    eflat = jnp.concatenate([e8[:, 0], e8[:, 1]])
    xs, pos2d, bexp = _sort_gather(eflat, c0, c1, xf)
    ys = _mlp(bexp[:NBLK], xs, gate_W, up_W, down_W)
    out = _combine(ys, pos2d, w8[:, 0], w8[:, 1])
    return out.reshape(B, S, D)


# speedup vs baseline: 2.0426x; 1.0007x over previous
import functools

import jax
import jax.numpy as jnp
from jax import lax
from jax.experimental import pallas as pl
from jax.experimental.pallas import tpu as pltpu
from jax.experimental.pallas import tpu_sc as plsc

E = 8
N = 4096
D = 1024
H = 2048
A = 2 * N
BM = 128
PAD = A + E * BM
NBLK = PAD // BM
NW = 32
APW = A // NW
TPW = N // NW
_NC = 2


def _router_body(x_ref, rw_ref, e_ref, w_ref, c0_ref, c1_ref):
    logits = jnp.dot(x_ref[...], rw_ref[...], preferred_element_type=jnp.float32)
    lane = lax.broadcasted_iota(jnp.int32, logits.shape, 1)
    m1 = jnp.max(logits, axis=1, keepdims=True)
    ft = jnp.min(jnp.where(logits == m1, lane, E), axis=1, keepdims=True)
    masked = jnp.where(lane == ft, -jnp.inf, logits)
    m2 = jnp.max(masked, axis=1, keepdims=True)
    st = jnp.min(jnp.where(masked == m2, lane, E), axis=1, keepdims=True)
    z = jnp.exp(m2 - m1)
    w1 = 1.0 / (1.0 + z)
    w2 = z / (1.0 + z)
    e_ref[...] = jnp.where(lane == 0, ft, jnp.where(lane == 1, st, 0))
    w_ref[...] = jnp.where(lane == 0, w1, jnp.where(lane == 1, w2, 0.0))
    h0 = jnp.sum(jnp.where(lane == ft, 1, 0), axis=0, keepdims=True)
    h1 = jnp.sum(jnp.where(lane == st, 1, 0), axis=0, keepdims=True)
    zpad = jnp.zeros((1, 16 - E), jnp.int32)
    c0_ref[...] = jnp.concatenate([h0, zpad], axis=1)[None]
    c1_ref[...] = jnp.concatenate([h1, zpad], axis=1)[None]


def _router(xf, router_W):
    TB = 256
    return pl.pallas_call(
        _router_body,
        grid=(N // TB,),
        in_specs=[
            pl.BlockSpec((TB, D), lambda i: (i, 0)),
            pl.BlockSpec((D, E), lambda i: (0, 0)),
        ],
        out_specs=[
            pl.BlockSpec((TB, E), lambda i: (i, 0)),
            pl.BlockSpec((TB, E), lambda i: (i, 0)),
            pl.BlockSpec((1, 1, 16), lambda i: (i, 0, 0)),
            pl.BlockSpec((1, 1, 16), lambda i: (i, 0, 0)),
        ],
        out_shape=[
            jax.ShapeDtypeStruct((N, E), jnp.int32),
            jax.ShapeDtypeStruct((N, E), jnp.float32),
            jax.ShapeDtypeStruct((NW // 2, 1, 16), jnp.int32),
            jax.ShapeDtypeStruct((NW // 2, 1, 16), jnp.int32),
        ],
    )(xf, router_W)


def _sort_gather_body(eflat, c0_hbm, c1_hbm, x_hbm, xs_hbm, pos_hbm, bexp_hbm,
                      eids_v, pos_mat, ac0_v, ac1_v, xbufa, xbufb,
                      bexp_v, sh_v, gsema, gsemb, ssema, ssemb):
    wid = lax.axis_index("s") * _NC + lax.axis_index("c")
    iota16 = lax.iota(jnp.int32, 16)
    z16 = jnp.zeros((16,), jnp.int32)

    pltpu.sync_copy(eflat.at[pl.ds(wid * APW, APW)], eids_v)
    pltpu.sync_copy(c0_hbm, ac0_v)
    pltpu.sync_copy(c1_hbm, ac1_v)

    sh_v[pl.ds(0, 16)] = z16
    sh_v[pl.ds(32, 16)] = z16

    def shift_cumsum(t):
        for k in (1, 2, 4, 8):
            sh_v[pl.ds(16, 16)] = t
            t = t + sh_v[pl.ds(16 - k, 16)]
        return t

    def splat_lane(t, e):
        sh_v[pl.ds(16, 16)] = t
        a = sh_v[pl.ds(16 + e, 16)]
        u = jnp.where(iota16 == 0, a, 0)
        for k in (1, 2, 4, 8):
            sh_v[pl.ds(16, 16)] = u
            u = u + sh_v[pl.ds(16 - k, 16)]
        return u

    cnts = [z16] * E
    for s in range(APW // 16):
        v = eids_v[pl.ds(s * 16, 16)]
        for e in range(E):
            cnts[e] = cnts[e] + jnp.where(v == e, 1, 0)
    lanepref = [shift_cumsum(c) - c for c in cnts]

    tot = z16
    pre = z16
    for t in range(NW):
        row = ac0_v[t, 0] if t < NW // 2 else ac1_v[t - NW // 2, 0]
        tot = tot + row
        flag = ((jnp.int32(t) - wid) >> 31) & 1
        pre = pre + row * flag

    rt = ((tot + BM - 1) >> 7) << 7
    base = shift_cumsum(rt) - rt
    startv = base + pre

    runs = []
    for e in range(E):
        runs.append(splat_lane(startv, e) + lanepref[e])

    for s in range(APW // 16):
        v = eids_v[pl.ds(s * 16, 16)]
        pos = z16
        for e in range(E):
            m = v == e
            pos = pos + jnp.where(m, runs[e], 0)
            runs[e] = runs[e] + jnp.where(m, 1, 0)
        pos_mat[s // 2, pl.ds((s % 2) * 16, 16)] = pos

    pltpu.sync_copy(pos_mat, pos_hbm.at[pl.ds(wid * 8, 8)])

    bufs = (xbufa, xbufb)
    gsems = (gsema, gsemb)
    ssems = (ssema, ssemb)
    NCH = APW // 32

    def xsrc(c):
        off = pl.multiple_of((wid * APW + c * 32) & (N - 1), 32)
        return x_hbm.at[pl.ds(off, 32)]

    g = {}
    sc = {}
    g[0] = pltpu.async_copy(xsrc(0), bufs[0], gsems[0])
    for c in range(NCH):
        if c + 1 < NCH:
            if c >= 1:
                sc[c - 1].wait()
            g[c + 1] = pltpu.async_copy(xsrc(c + 1), bufs[(c + 1) % 2],
                                        gsems[(c + 1) % 2])
        g[c].wait()
        sc[c] = pltpu.async_copy(
            bufs[c % 2], xs_hbm.at[pos_mat.at[c]], ssems[c % 2])
    sc[NCH - 2].wait()
    sc[NCH - 1].wait()

    @pl.when(wid == 0)
    def _():
        bes = [splat_lane(base, e) for e in range(1, E)]
        for j in range(5):
            bs = (j * 16 + iota16) << 7
            expv = z16
            for be in bes:
                expv = expv + jnp.where(bs >= be, 1, 0)
            bexp_v[pl.ds(j * 16, 16)] = expv
        pltpu.sync_copy(bexp_v, bexp_hbm)


def _sort_gather(*args):
    return functools.partial(
        pl.kernel,
        out_type=[
            jax.ShapeDtypeStruct((PAD, D), jnp.float32),
            jax.ShapeDtypeStruct((A // 32, 32), jnp.int32),
            jax.ShapeDtypeStruct((80,), jnp.int32),
        ],
        mesh=plsc.VectorSubcoreMesh(core_axis_name="c", subcore_axis_name="s"),
        scratch_types=[
            pltpu.VMEM((APW,), jnp.int32),
            pltpu.VMEM((8, 32), jnp.int32),
            pltpu.VMEM((NW // 2, 1, 16), jnp.int32),
            pltpu.VMEM((NW // 2, 1, 16), jnp.int32),
            pltpu.VMEM((32, D), jnp.float32),
            pltpu.VMEM((32, D), jnp.float32),
            pltpu.VMEM((80,), jnp.int32),
            pltpu.VMEM((48,), jnp.int32),
            pltpu.SemaphoreType.DMA,
            pltpu.SemaphoreType.DMA,
            pltpu.SemaphoreType.DMA,
            pltpu.SemaphoreType.DMA,
        ],
    )(_sort_gather_body)(*args)


def _mlp_body(be_ref, xs_ref, gw_ref, uw_ref, dw_ref, ys_ref):
    xc = xs_ref[...]
    g = jnp.dot(xc, gw_ref[0], preferred_element_type=jnp.float32)
    g = g * jax.nn.sigmoid(g)
    u = jnp.dot(xc, uw_ref[0], preferred_element_type=jnp.float32)
    ys_ref[...] = jnp.dot(g * u, dw_ref[0], preferred_element_type=jnp.float32)


def _mlp(bexp, xs, gate_W, up_W, down_W):
    grid_spec = pltpu.PrefetchScalarGridSpec(
        num_scalar_prefetch=1,
        grid=(NBLK,),
        in_specs=[
            pl.BlockSpec((BM, D), lambda b, be: (b, 0)),
            pl.BlockSpec((1, D, H), lambda b, be: (be[b], 0, 0)),
            pl.BlockSpec((1, D, H), lambda b, be: (be[b], 0, 0)),
            pl.BlockSpec((1, H, D), lambda b, be: (be[b], 0, 0)),
        ],
        out_specs=pl.BlockSpec((BM, D), lambda b, be: (b, 0)),
    )
    return pl.pallas_call(
        _mlp_body,
        grid_spec=grid_spec,
        out_shape=jax.ShapeDtypeStruct((PAD, D), jnp.float32),
    )(bexp, xs, gate_W, up_W, down_W)


def _combine_body(ys_hbm, pos_hbm, w0_hbm, w1_hbm, out_hbm,
                  p0_mat, p1_mat, w0_v, w1_v, shf_v,
                  yb0a, yb1a, oba, yb0b, yb1b, obb,
                  g0a, g1a, wsa, g0b, g1b, wsb):
    wid = lax.axis_index("s") * _NC + lax.axis_index("c")
    iota16 = lax.iota(jnp.int32, 16)
    zf16 = jnp.zeros((16,), jnp.float32)
    t0 = wid * TPW

    pltpu.sync_copy(pos_hbm.at[pl.ds(4 * wid, 4)], p0_mat)
    pltpu.sync_copy(pos_hbm.at[pl.ds(128 + 4 * wid, 4)], p1_mat)
    pltpu.sync_copy(w0_hbm.at[pl.ds(t0, TPW)], w0_v.at[pl.ds(0, TPW)])
    pltpu.sync_copy(w1_hbm.at[pl.ds(t0, TPW)], w1_v.at[pl.ds(0, TPW)])
    w0_v[pl.ds(TPW, 16)] = zf16
    w1_v[pl.ds(TPW, 16)] = zf16
    shf_v[pl.ds(0, 16)] = zf16
    shf_v[pl.ds(32, 16)] = zf16

    sets = ((yb0a, yb1a, oba, g0a, g1a, wsa),
            (yb0b, yb1b, obb, g0b, g1b, wsb))

    def splat(ref, ri):
        a = ref[pl.ds(ri, 16)]
        u = jnp.where(iota16 == 0, a, 0.0)
        for k in (1, 2, 4, 8):
            shf_v[pl.ds(16, 16)] = u
            u = u + shf_v[pl.ds(16 - k, 16)]
        return u

    NCH = TPW // 16

    def idx0(c):
        return p0_mat.at[c // 2, pl.ds((c % 2) * 16, 16)]

    def idx1(c):
        return p1_mat.at[c // 2, pl.ds((c % 2) * 16, 16)]

    h = {}
    yb0, yb1, ob, sg0, sg1, sw = sets[0]
    h[(0, 0)] = pltpu.async_copy(ys_hbm.at[idx0(0)], yb0, sg0)
    h[(0, 1)] = pltpu.async_copy(ys_hbm.at[idx1(0)], yb1, sg1)
    for c in range(NCH):
        yb0, yb1, ob, sg0, sg1, sw = sets[c % 2]
        if c + 1 < NCH:
            nb0, nb1, nob, ng0, ng1, nw = sets[(c + 1) % 2]
            if c >= 1:
                h[(c - 1, "w")].wait()
            h[(c + 1, 0)] = pltpu.async_copy(ys_hbm.at[idx0(c + 1)], nb0, ng0)
            h[(c + 1, 1)] = pltpu.async_copy(ys_hbm.at[idx1(c + 1)], nb1, ng1)
        h[(c, 0)].wait()
        h[(c, 1)].wait()

        def row_body(r, _, yb0=yb0, yb1=yb1, ob=ob, c=c):
            ri = c * 16 + r
            w0s = splat(w0_v, ri)
            w1s = splat(w1_v, ri)
            for q in range(D // 16):
                sl = pl.ds(q * 16, 16)
                ob[r, sl] = yb0[r, sl] * w0s + yb1[r, sl] * w1s
            return 0

        lax.fori_loop(0, 16, row_body, 0)
        h[(c, "w")] = pltpu.async_copy(
            ob, out_hbm.at[pl.ds(t0 + c * 16, 16)], sw)
    h[(NCH - 2, "w")].wait()
    h[(NCH - 1, "w")].wait()


def _combine(*args):
    return functools.partial(
        pl.kernel,
        out_type=jax.ShapeDtypeStruct((N, D), jnp.float32),
        mesh=plsc.VectorSubcoreMesh(core_axis_name="c", subcore_axis_name="s"),
        scratch_types=[
            pltpu.VMEM((4, 32), jnp.int32),
            pltpu.VMEM((4, 32), jnp.int32),
            pltpu.VMEM((TPW + 16,), jnp.float32),
            pltpu.VMEM((TPW + 16,), jnp.float32),
            pltpu.VMEM((48,), jnp.float32),
            pltpu.VMEM((16, D), jnp.float32),
            pltpu.VMEM((16, D), jnp.float32),
            pltpu.VMEM((16, D), jnp.float32),
            pltpu.VMEM((16, D), jnp.float32),
            pltpu.VMEM((16, D), jnp.float32),
            pltpu.VMEM((16, D), jnp.float32),
            pltpu.SemaphoreType.DMA,
            pltpu.SemaphoreType.DMA,
            pltpu.SemaphoreType.DMA,
            pltpu.SemaphoreType.DMA,
            pltpu.SemaphoreType.DMA,
            pltpu.SemaphoreType.DMA,
        ],
    )(_combine_body)(*args)


def kernel(x, router_W, gate_W, up_W, down_W):
    B, S, _ = x.shape
    xf = x.reshape(-1, D)
    e8, w8, c0, c1 = _router(xf, router_W)
    eflat = jnp.concatenate([e8[:, 0], e8[:, 1]])
    xs, pos2d, bexp = _sort_gather(eflat, c0, c1, xf)
    ys = _mlp(bexp[:NBLK], xs, gate_W, up_W, down_W)
    out = _combine(ys, pos2d, w8[:, 0], w8[:, 1])
    return out.reshape(B, S, D)


# speedup vs baseline: 2.0427x; 1.0001x over previous
import functools

import jax
import jax.numpy as jnp
from jax import lax
from jax.experimental import pallas as pl
from jax.experimental.pallas import tpu as pltpu
from jax.experimental.pallas import tpu_sc as plsc

E = 8
N = 4096
D = 1024
H = 2048
A = 2 * N
BM = 128
PAD = A + E * BM
NBLK = PAD // BM
NW = 32
APW = A // NW
TPW = N // NW
_NC = 2


def _router_body(x_ref, rw_ref, e_ref, w_ref, c0_ref, c1_ref):
    logits = jnp.dot(x_ref[...], rw_ref[...], preferred_element_type=jnp.float32)
    lane = lax.broadcasted_iota(jnp.int32, logits.shape, 1)
    m1 = jnp.max(logits, axis=1, keepdims=True)
    ft = jnp.min(jnp.where(logits == m1, lane, E), axis=1, keepdims=True)
    masked = jnp.where(lane == ft, -jnp.inf, logits)
    m2 = jnp.max(masked, axis=1, keepdims=True)
    st = jnp.min(jnp.where(masked == m2, lane, E), axis=1, keepdims=True)
    z = jnp.exp(m2 - m1)
    w1 = 1.0 / (1.0 + z)
    w2 = z / (1.0 + z)
    e_ref[...] = jnp.where(lane == 0, ft, jnp.where(lane == 1, st, 0))
    w_ref[...] = jnp.where(lane == 0, w1, jnp.where(lane == 1, w2, 0.0))
    h0 = jnp.sum(jnp.where(lane == ft, 1, 0), axis=0, keepdims=True)
    h1 = jnp.sum(jnp.where(lane == st, 1, 0), axis=0, keepdims=True)
    zpad = jnp.zeros((1, 16 - E), jnp.int32)
    c0_ref[...] = jnp.concatenate([h0, zpad], axis=1)[None]
    c1_ref[...] = jnp.concatenate([h1, zpad], axis=1)[None]


def _router(xf, router_W):
    TB = 256
    return pl.pallas_call(
        _router_body,
        grid=(N // TB,),
        in_specs=[
            pl.BlockSpec((TB, D), lambda i: (i, 0)),
            pl.BlockSpec((D, E), lambda i: (0, 0)),
        ],
        out_specs=[
            pl.BlockSpec((TB, E), lambda i: (i, 0)),
            pl.BlockSpec((TB, E), lambda i: (i, 0)),
            pl.BlockSpec((1, 1, 16), lambda i: (i, 0, 0)),
            pl.BlockSpec((1, 1, 16), lambda i: (i, 0, 0)),
        ],
        out_shape=[
            jax.ShapeDtypeStruct((N, E), jnp.int32),
            jax.ShapeDtypeStruct((N, E), jnp.float32),
            jax.ShapeDtypeStruct((NW // 2, 1, 16), jnp.int32),
            jax.ShapeDtypeStruct((NW // 2, 1, 16), jnp.int32),
        ],
    )(xf, router_W)


def _sort_gather_body(eflat, c0_hbm, c1_hbm, x_hbm, xs_hbm, pos_hbm, bexp_hbm,
                      eids_v, pos_mat, ac0_v, ac1_v, xbufa, xbufb,
                      bexp_v, sh_v, gsema, gsemb, ssema, ssemb):
    wid = lax.axis_index("s") * _NC + lax.axis_index("c")
    iota16 = lax.iota(jnp.int32, 16)
    z16 = jnp.zeros((16,), jnp.int32)

    pltpu.sync_copy(eflat.at[pl.ds(wid * APW, APW)], eids_v)
    pltpu.sync_copy(c0_hbm, ac0_v)
    pltpu.sync_copy(c1_hbm, ac1_v)

    sh_v[pl.ds(0, 16)] = z16
    sh_v[pl.ds(32, 16)] = z16

    def shift_cumsum(t):
        for k in (1, 2, 4, 8):
            sh_v[pl.ds(16, 16)] = t
            t = t + sh_v[pl.ds(16 - k, 16)]
        return t

    def splat_lane(t, e):
        sh_v[pl.ds(16, 16)] = t
        a = sh_v[pl.ds(16 + e, 16)]
        u = jnp.where(iota16 == 0, a, 0)
        for k in (1, 2, 4, 8):
            sh_v[pl.ds(16, 16)] = u
            u = u + sh_v[pl.ds(16 - k, 16)]
        return u

    cnts = [z16] * E
    for s in range(APW // 16):
        v = eids_v[pl.ds(s * 16, 16)]
        for e in range(E):
            cnts[e] = cnts[e] + jnp.where(v == e, 1, 0)
    lanepref = [shift_cumsum(c) - c for c in cnts]

    tot = z16
    pre = z16
    for t in range(NW):
        row = ac0_v[t, 0] if t < NW // 2 else ac1_v[t - NW // 2, 0]
        tot = tot + row
        flag = ((jnp.int32(t) - wid) >> 31) & 1
        pre = pre + row * flag

    rt = ((tot + BM - 1) >> 7) << 7
    base = shift_cumsum(rt) - rt
    startv = base + pre

    runs = []
    for e in range(E):
        runs.append(splat_lane(startv, e) + lanepref[e])

    for s in range(APW // 16):
        v = eids_v[pl.ds(s * 16, 16)]
        pos = z16
        for e in range(E):
            m = v == e
            pos = pos + jnp.where(m, runs[e], 0)
            runs[e] = runs[e] + jnp.where(m, 1, 0)
        pos_mat[s // 2, pl.ds((s % 2) * 16, 16)] = pos

    pltpu.sync_copy(pos_mat, pos_hbm.at[pl.ds(wid * 8, 8)])

    bufs = (xbufa, xbufb)
    gsems = (gsema, gsemb)
    ssems = (ssema, ssemb)
    NCH = APW // 32

    def xsrc(c):
        off = pl.multiple_of((wid * APW + c * 32) & (N - 1), 32)
        return x_hbm.at[pl.ds(off, 32)]

    g = {}
    sc = {}
    g[0] = pltpu.async_copy(xsrc(0), bufs[0], gsems[0])
    for c in range(NCH):
        if c + 1 < NCH:
            if c >= 1:
                sc[c - 1].wait()
            g[c + 1] = pltpu.async_copy(xsrc(c + 1), bufs[(c + 1) % 2],
                                        gsems[(c + 1) % 2])
        g[c].wait()
        sc[c] = pltpu.async_copy(
            bufs[c % 2], xs_hbm.at[pos_mat.at[c]], ssems[c % 2])
    sc[NCH - 2].wait()
    sc[NCH - 1].wait()

    @pl.when(wid == 0)
    def _():
        bes = [splat_lane(base, e) for e in range(1, E)]
        for j in range(5):
            bs = (j * 16 + iota16) << 7
            expv = z16
            for be in bes:
                expv = expv + jnp.where(bs >= be, 1, 0)
            bexp_v[pl.ds(j * 16, 16)] = expv
        pltpu.sync_copy(bexp_v, bexp_hbm)


def _sort_gather(*args):
    return functools.partial(
        pl.kernel,
        out_type=[
            jax.ShapeDtypeStruct((PAD, D), jnp.float32),
            jax.ShapeDtypeStruct((A // 32, 32), jnp.int32),
            jax.ShapeDtypeStruct((80,), jnp.int32),
        ],
        mesh=plsc.VectorSubcoreMesh(core_axis_name="c", subcore_axis_name="s"),
        scratch_types=[
            pltpu.VMEM((APW,), jnp.int32),
            pltpu.VMEM((8, 32), jnp.int32),
            pltpu.VMEM((NW // 2, 1, 16), jnp.int32),
            pltpu.VMEM((NW // 2, 1, 16), jnp.int32),
            pltpu.VMEM((32, D), jnp.float32),
            pltpu.VMEM((32, D), jnp.float32),
            pltpu.VMEM((80,), jnp.int32),
            pltpu.VMEM((48,), jnp.int32),
            pltpu.SemaphoreType.DMA,
            pltpu.SemaphoreType.DMA,
            pltpu.SemaphoreType.DMA,
            pltpu.SemaphoreType.DMA,
        ],
    )(_sort_gather_body)(*args)


def _mlp_body(be_ref, xs_ref, gw_ref, uw_ref, dw_ref, ys_ref):
    xc = xs_ref[...]
    g = jnp.dot(xc, gw_ref[0], preferred_element_type=jnp.float32)
    g = g * jax.nn.sigmoid(g)
    u = jnp.dot(xc, uw_ref[0], preferred_element_type=jnp.float32)
    ys_ref[...] = jnp.dot(g * u, dw_ref[0], preferred_element_type=jnp.float32)


def _mlp(bexp, xs, gate_W, up_W, down_W):
    grid_spec = pltpu.PrefetchScalarGridSpec(
        num_scalar_prefetch=1,
        grid=(NBLK,),
        in_specs=[
            pl.BlockSpec((BM, D), lambda b, be: (b, 0)),
            pl.BlockSpec((1, D, H), lambda b, be: (be[b], 0, 0)),
            pl.BlockSpec((1, D, H), lambda b, be: (be[b], 0, 0)),
            pl.BlockSpec((1, H, D), lambda b, be: (be[b], 0, 0)),
        ],
        out_specs=pl.BlockSpec((BM, D), lambda b, be: (b, 0)),
    )
    return pl.pallas_call(
        _mlp_body,
        grid_spec=grid_spec,
        out_shape=jax.ShapeDtypeStruct((PAD, D), jnp.float32),
        compiler_params=pltpu.CompilerParams(
            dimension_semantics=("parallel",)),
    )(bexp, xs, gate_W, up_W, down_W)


def _combine_body(ys_hbm, pos_hbm, w0_hbm, w1_hbm, out_hbm,
                  p0_mat, p1_mat, w0_v, w1_v, shf_v,
                  yb0a, yb1a, oba, yb0b, yb1b, obb,
                  g0a, g1a, wsa, g0b, g1b, wsb):
    wid = lax.axis_index("s") * _NC + lax.axis_index("c")
    iota16 = lax.iota(jnp.int32, 16)
    zf16 = jnp.zeros((16,), jnp.float32)
    t0 = wid * TPW

    pltpu.sync_copy(pos_hbm.at[pl.ds(4 * wid, 4)], p0_mat)
    pltpu.sync_copy(pos_hbm.at[pl.ds(128 + 4 * wid, 4)], p1_mat)
    pltpu.sync_copy(w0_hbm.at[pl.ds(t0, TPW)], w0_v.at[pl.ds(0, TPW)])
    pltpu.sync_copy(w1_hbm.at[pl.ds(t0, TPW)], w1_v.at[pl.ds(0, TPW)])
    w0_v[pl.ds(TPW, 16)] = zf16
    w1_v[pl.ds(TPW, 16)] = zf16
    shf_v[pl.ds(0, 16)] = zf16
    shf_v[pl.ds(32, 16)] = zf16

    sets = ((yb0a, yb1a, oba, g0a, g1a, wsa),
            (yb0b, yb1b, obb, g0b, g1b, wsb))

    def splat(ref, ri):
        a = ref[pl.ds(ri, 16)]
        u = jnp.where(iota16 == 0, a, 0.0)
        for k in (1, 2, 4, 8):
            shf_v[pl.ds(16, 16)] = u
            u = u + shf_v[pl.ds(16 - k, 16)]
        return u

    NCH = TPW // 16

    def idx0(c):
        return p0_mat.at[c // 2, pl.ds((c % 2) * 16, 16)]

    def idx1(c):
        return p1_mat.at[c // 2, pl.ds((c % 2) * 16, 16)]

    h = {}
    yb0, yb1, ob, sg0, sg1, sw = sets[0]
    h[(0, 0)] = pltpu.async_copy(ys_hbm.at[idx0(0)], yb0, sg0)
    h[(0, 1)] = pltpu.async_copy(ys_hbm.at[idx1(0)], yb1, sg1)
    for c in range(NCH):
        yb0, yb1, ob, sg0, sg1, sw = sets[c % 2]
        if c + 1 < NCH:
            nb0, nb1, nob, ng0, ng1, nw = sets[(c + 1) % 2]
            if c >= 1:
                h[(c - 1, "w")].wait()
            h[(c + 1, 0)] = pltpu.async_copy(ys_hbm.at[idx0(c + 1)], nb0, ng0)
            h[(c + 1, 1)] = pltpu.async_copy(ys_hbm.at[idx1(c + 1)], nb1, ng1)
        h[(c, 0)].wait()
        h[(c, 1)].wait()

        def row_body(r, _, yb0=yb0, yb1=yb1, ob=ob, c=c):
            ri = c * 16 + r
            w0s = splat(w0_v, ri)
            w1s = splat(w1_v, ri)
            for q in range(D // 16):
                sl = pl.ds(q * 16, 16)
                ob[r, sl] = yb0[r, sl] * w0s + yb1[r, sl] * w1s
            return 0

        lax.fori_loop(0, 16, row_body, 0)
        h[(c, "w")] = pltpu.async_copy(
            ob, out_hbm.at[pl.ds(t0 + c * 16, 16)], sw)
    h[(NCH - 2, "w")].wait()
    h[(NCH - 1, "w")].wait()


def _combine(*args):
    return functools.partial(
        pl.kernel,
        out_type=jax.ShapeDtypeStruct((N, D), jnp.float32),
        mesh=plsc.VectorSubcoreMesh(core_axis_name="c", subcore_axis_name="s"),
        scratch_types=[
            pltpu.VMEM((4, 32), jnp.int32),
            pltpu.VMEM((4, 32), jnp.int32),
            pltpu.VMEM((TPW + 16,), jnp.float32),
            pltpu.VMEM((TPW + 16,), jnp.float32),
            pltpu.VMEM((48,), jnp.float32),
            pltpu.VMEM((16, D), jnp.float32),
            pltpu.VMEM((16, D), jnp.float32),
            pltpu.VMEM((16, D), jnp.float32),
            pltpu.VMEM((16, D), jnp.float32),
            pltpu.VMEM((16, D), jnp.float32),
            pltpu.VMEM((16, D), jnp.float32),
            pltpu.SemaphoreType.DMA,
            pltpu.SemaphoreType.DMA,
            pltpu.SemaphoreType.DMA,
            pltpu.SemaphoreType.DMA,
            pltpu.SemaphoreType.DMA,
            pltpu.SemaphoreType.DMA,
        ],
    )(_combine_body)(*args)


def kernel(x, router_W, gate_W, up_W, down_W):
    B, S, _ = x.shape
    xf = x.reshape(-1, D)
    e8, w8, c0, c1 = _router(xf, router_W)
    eflat = jnp.concatenate([e8[:, 0], e8[:, 1]])
    xs, pos2d, bexp = _sort_gather(eflat, c0, c1, xf)
    ys = _mlp(bexp[:NBLK], xs, gate_W, up_W, down_W)
    out = _combine(ys, pos2d, w8[:, 0], w8[:, 1])
    return out.reshape(B, S, D)
